# Initial kernel scaffold; baseline (speedup 1.0000x reference)
import jax
import jax.numpy as jnp
from jax.experimental import pallas as pl


def kernel(pred, target):
    raise NotImplementedError("write your pallas kernel here")


# speedup vs baseline: 1.7174x; 1.7174x over previous
import functools

import jax
import jax.numpy as jnp
from jax.experimental import pallas as pl

_SMOOTHING = 0.1
_IGNORE_INDEX = 0
_CONFIDENCE = 1.0 - _SMOOTHING


def _row_stats_kernel(pred_ref, tgt_ref, pt_ref, keep_ref, *, ncls):
    x = pred_ref[:, :]
    t = tgt_ref[:, :]
    br, c = x.shape

    m = jnp.max(x, axis=1, keepdims=True)
    s = jnp.sum(jnp.exp(x - m), axis=1, keepdims=True)
    lse = m + jnp.log(s)
    tot = jnp.sum(x, axis=1, keepdims=True)

    col = jax.lax.broadcasted_iota(jnp.int32, (br, c), 1)
    xt = jnp.sum(jnp.where(col == t, x, 0.0), axis=1, keepdims=True)

    fill = _SMOOTHING / (ncls - 1) if ncls > 1 else _SMOOTHING
    pt = -(fill * (tot - ncls * lse) + (_CONFIDENCE - fill) * (xt - lse))
    keep = (t != _IGNORE_INDEX).astype(x.dtype)
    pt_ref[:, :] = pt * keep
    keep_ref[:, :] = keep


def kernel(pred, target):
    n, c = pred.shape
    br = 8
    tgt2d = target.astype(jnp.int32).reshape(n, 1)
    pt, keep = pl.pallas_call(
        functools.partial(_row_stats_kernel, ncls=c),
        grid=(n // br,),
        in_specs=[
            pl.BlockSpec((br, c), lambda i: (i, 0)),
            pl.BlockSpec((br, 1), lambda i: (i, 0)),
        ],
        out_specs=[
            pl.BlockSpec((br, 1), lambda i: (i, 0)),
            pl.BlockSpec((br, 1), lambda i: (i, 0)),
        ],
        out_shape=[
            jax.ShapeDtypeStruct((n, 1), pred.dtype),
            jax.ShapeDtypeStruct((n, 1), pred.dtype),
        ],
    )(pred, tgt2d)
    return jnp.sum(pt) / jnp.maximum(jnp.sum(keep), 1.0)


# speedup vs baseline: 1.9470x; 1.1337x over previous
import functools

import jax
import jax.numpy as jnp
from jax.experimental import pallas as pl

_SMOOTHING = 0.1
_IGNORE_INDEX = 0
_CONFIDENCE = 1.0 - _SMOOTHING


def _row_stats_kernel(pred_ref, tgt_ref, pt_ref, keep_ref, *, ncls):
    x = pred_ref[:, :]
    t = tgt_ref[:, :]
    br, c = x.shape

    m = jnp.max(x, axis=1, keepdims=True)
    s = jnp.sum(jnp.exp(x - m), axis=1, keepdims=True)
    lse = m + jnp.log(s)
    tot = jnp.sum(x, axis=1, keepdims=True)

    col = jax.lax.broadcasted_iota(jnp.int32, (br, c), 1)
    xt = jnp.sum(jnp.where(col == t, x, 0.0), axis=1, keepdims=True)

    fill = _SMOOTHING / (ncls - 1) if ncls > 1 else _SMOOTHING
    pt = -(fill * (tot - ncls * lse) + (_CONFIDENCE - fill) * (xt - lse))
    keep = (t != _IGNORE_INDEX).astype(x.dtype)
    pt_ref[:, :] = pt * keep
    keep_ref[:, :] = keep


def kernel(pred, target):
    n, c = pred.shape
    br = 16
    tgt2d = target.astype(jnp.int32).reshape(n, 1)
    pt, keep = pl.pallas_call(
        functools.partial(_row_stats_kernel, ncls=c),
        grid=(n // br,),
        in_specs=[
            pl.BlockSpec((br, c), lambda i: (i, 0)),
            pl.BlockSpec((br, 1), lambda i: (i, 0)),
        ],
        out_specs=[
            pl.BlockSpec((br, 1), lambda i: (i, 0)),
            pl.BlockSpec((br, 1), lambda i: (i, 0)),
        ],
        out_shape=[
            jax.ShapeDtypeStruct((n, 1), pred.dtype),
            jax.ShapeDtypeStruct((n, 1), pred.dtype),
        ],
    )(pred, tgt2d)
    return jnp.sum(pt) / jnp.maximum(jnp.sum(keep), 1.0)


# speedup vs baseline: 2.0700x; 1.0632x over previous
import functools

import jax
import jax.numpy as jnp
from jax.experimental import pallas as pl

_SMOOTHING = 0.1
_IGNORE_INDEX = 0
_CONFIDENCE = 1.0 - _SMOOTHING


def _row_stats_kernel(pred_ref, tgt_ref, pt_ref, keep_ref, *, ncls):
    x = pred_ref[:, :]
    t = tgt_ref[:, :]
    br, c = x.shape

    m = jnp.max(x, axis=1, keepdims=True)
    s = jnp.sum(jnp.exp(x - m), axis=1, keepdims=True)
    lse = m + jnp.log(s)
    tot = jnp.sum(x, axis=1, keepdims=True)

    col = jax.lax.broadcasted_iota(jnp.int32, (br, c), 1)
    xt = jnp.sum(jnp.where(col == t, x, 0.0), axis=1, keepdims=True)

    fill = _SMOOTHING / (ncls - 1) if ncls > 1 else _SMOOTHING
    pt = -(fill * (tot - ncls * lse) + (_CONFIDENCE - fill) * (xt - lse))
    keep = (t != _IGNORE_INDEX).astype(x.dtype)
    pt_ref[:, :] = pt * keep
    keep_ref[:, :] = keep


def kernel(pred, target):
    n, c = pred.shape
    br = 32
    tgt2d = target.astype(jnp.int32).reshape(n, 1)
    pt, keep = pl.pallas_call(
        functools.partial(_row_stats_kernel, ncls=c),
        grid=(n // br,),
        in_specs=[
            pl.BlockSpec((br, c), lambda i: (i, 0)),
            pl.BlockSpec((br, 1), lambda i: (i, 0)),
        ],
        out_specs=[
            pl.BlockSpec((br, 1), lambda i: (i, 0)),
            pl.BlockSpec((br, 1), lambda i: (i, 0)),
        ],
        out_shape=[
            jax.ShapeDtypeStruct((n, 1), pred.dtype),
            jax.ShapeDtypeStruct((n, 1), pred.dtype),
        ],
    )(pred, tgt2d)
    return jnp.sum(pt) / jnp.maximum(jnp.sum(keep), 1.0)


# speedup vs baseline: 2.1331x; 1.0305x over previous
import functools

import jax
import jax.numpy as jnp
from jax.experimental import pallas as pl

_SMOOTHING = 0.1
_IGNORE_INDEX = 0
_CONFIDENCE = 1.0 - _SMOOTHING


def _row_stats_kernel(pred_ref, tgt_ref, pt_ref, keep_ref, *, ncls):
    x = pred_ref[:, :]
    t = tgt_ref[:, :]
    br, c = x.shape

    m = jnp.max(x, axis=1, keepdims=True)
    s = jnp.sum(jnp.exp(x - m), axis=1, keepdims=True)
    lse = m + jnp.log(s)
    tot = jnp.sum(x, axis=1, keepdims=True)

    col = jax.lax.broadcasted_iota(jnp.int32, (br, c), 1)
    xt = jnp.sum(jnp.where(col == t, x, 0.0), axis=1, keepdims=True)

    fill = _SMOOTHING / (ncls - 1) if ncls > 1 else _SMOOTHING
    pt = -(fill * (tot - ncls * lse) + (_CONFIDENCE - fill) * (xt - lse))
    keep = (t != _IGNORE_INDEX).astype(x.dtype)
    pt_ref[:, :] = pt * keep
    keep_ref[:, :] = keep


def kernel(pred, target):
    n, c = pred.shape
    br = 64
    tgt2d = target.astype(jnp.int32).reshape(n, 1)
    pt, keep = pl.pallas_call(
        functools.partial(_row_stats_kernel, ncls=c),
        grid=(n // br,),
        in_specs=[
            pl.BlockSpec((br, c), lambda i: (i, 0)),
            pl.BlockSpec((br, 1), lambda i: (i, 0)),
        ],
        out_specs=[
            pl.BlockSpec((br, 1), lambda i: (i, 0)),
            pl.BlockSpec((br, 1), lambda i: (i, 0)),
        ],
        out_shape=[
            jax.ShapeDtypeStruct((n, 1), pred.dtype),
            jax.ShapeDtypeStruct((n, 1), pred.dtype),
        ],
    )(pred, tgt2d)
    return jnp.sum(pt) / jnp.maximum(jnp.sum(keep), 1.0)


# speedup vs baseline: 2.1621x; 1.0136x over previous
import functools

import jax
import jax.numpy as jnp
from jax import lax
from jax.experimental import pallas as pl
from jax.experimental.pallas import tpu as pltpu
from jax.experimental.pallas import tpu_sc as plsc

_SMOOTHING = 0.1
_IGNORE_INDEX = 0
_CONFIDENCE = 1.0 - _SMOOTHING

_N_SC = 256
_RPW = 8
_CC = 7680


def _row_stats_kernel(pred_ref, tgt_ref, pt_ref, keep_ref, *, ncls):
    x = pred_ref[:, :]
    t = tgt_ref[:, :]
    br, c = x.shape

    m = jnp.max(x, axis=1, keepdims=True)
    s = jnp.sum(jnp.exp(x - m), axis=1, keepdims=True)
    lse = m + jnp.log(s)
    tot = jnp.sum(x, axis=1, keepdims=True)

    col = jax.lax.broadcasted_iota(jnp.int32, (br, c), 1)
    xt = jnp.sum(jnp.where(col == t, x, 0.0), axis=1, keepdims=True)

    fill = _SMOOTHING / (ncls - 1) if ncls > 1 else _SMOOTHING
    pt = -(fill * (tot - ncls * lse) + (_CONFIDENCE - fill) * (xt - lse))
    keep = (t != _IGNORE_INDEX).astype(x.dtype)
    pt_ref[:, :] = pt * keep
    keep_ref[:, :] = keep


def _tail_stats_kernel(pred_ref, tgt_ref, m_ref, s_ref, tot_ref, xt_ref,
                       *, col0):
    x = pred_ref[:, :]
    t = tgt_ref[:, :]
    br, w = x.shape
    m = jnp.max(x, axis=1, keepdims=True)
    s = jnp.sum(jnp.exp(x - m), axis=1, keepdims=True)
    tot = jnp.sum(x, axis=1, keepdims=True)
    col = col0 + jax.lax.broadcasted_iota(jnp.int32, (br, w), 1)
    xt = jnp.sum(jnp.where(col == t, x, 0.0), axis=1, keepdims=True)
    m_ref[:, :] = m
    s_ref[:, :] = s
    tot_ref[:, :] = tot
    xt_ref[:, :] = xt


def _sc_stats_kernel(pred_hbm, tgt_hbm, out_hbm,
                     buf0, buf1, tgt_v, stats_v, staged,
                     sem0, sem1, semg, *, n_tc, c_sc, chunks):
    wid = lax.axis_index("c") * 16 + lax.axis_index("s")
    row0 = pl.multiple_of(n_tc + wid * _RPW, 8)
    n_chunks = len(chunks)
    bufs = (buf0, buf1)
    sems = (sem0, sem1)

    pltpu.sync_copy(tgt_hbm.at[pl.ds(row0, _RPW)], tgt_v.at[pl.ds(0, _RPW)])
    tv_all = tgt_v[...]

    def _start(step):
        c0, cc = chunks[step]
        pltpu.make_async_copy(
            pred_hbm.at[pl.ds(row0, _RPW), pl.ds(c0, cc)],
            bufs[step % 2].at[:, pl.ds(0, cc)], sems[step % 2]).start()

    iota16 = lax.iota(jnp.int32, 16)
    neg_inf = jnp.full((16,), -jnp.inf, jnp.float32)
    zeros = jnp.zeros((16,), jnp.float32)

    m_run = [neg_inf] * _RPW
    s_run = [zeros] * _RPW
    t_run = [zeros] * _RPW

    _start(0)
    for step in range(n_chunks):
        c0, cc = chunks[step]
        buf = bufs[step % 2]
        pltpu.make_async_copy(
            pred_hbm.at[pl.ds(row0, _RPW), pl.ds(c0, cc)],
            buf.at[:, pl.ds(0, cc)], sems[step % 2]).wait()
        if step + 1 < n_chunks:
            _start(step + 1)

        for r in range(_RPW):
            def _sweep1(i, carry):
                cm, tv = carry
                v = buf[r, pl.ds(i * 16, 16)]
                return jnp.maximum(cm, v), tv + v
            cm, t_run[r] = lax.fori_loop(
                0, cc // 16, _sweep1, (m_run[r], t_run[r]), unroll=4)
            s_run[r] = s_run[r] * jnp.exp(m_run[r] - cm)
            m_run[r] = cm

            def _sweep2(i, sv):
                v = buf[r, pl.ds(i * 16, 16)]
                return sv + jnp.exp(v - cm)
            s_run[r] = lax.fori_loop(
                0, cc // 16, _sweep2, s_run[r], unroll=4)

    for r in range(_RPW):
        t_s = tv_all[r]
        c_t = jnp.minimum((t_s // 128) * 128, c_sc - 128)
        c_t = pl.multiple_of(c_t, 128)
        pltpu.make_async_copy(
            pred_hbm.at[pl.ds(row0, _RPW), pl.ds(c_t, 128)],
            staged, semg).start()
        pltpu.make_async_copy(
            pred_hbm.at[pl.ds(row0, _RPW), pl.ds(c_t, 128)],
            staged, semg).wait()
        rel = jnp.full((16,), t_s - c_t, jnp.int32)
        xt = zeros
        for j in range(8):
            v = staged[r, pl.ds(j * 16, 16)]
            xt = xt + jnp.where(iota16 + j * 16 == rel, v, 0.0)
        base = r * 64
        stats_v[pl.ds(base, 16)] = m_run[r]
        stats_v[pl.ds(base + 16, 16)] = s_run[r]
        stats_v[pl.ds(base + 32, 16)] = t_run[r]
        stats_v[pl.ds(base + 48, 16)] = xt

    pltpu.sync_copy(stats_v, out_hbm.at[pl.ds(wid * _RPW * 64, _RPW * 64)])


def kernel(pred, target):
    n, c = pred.shape
    n_tc = n - _N_SC
    br = 64
    tgt = target.astype(jnp.int32)
    tgt2d = tgt.reshape(n, 1)

    c_sc = (c // 128) * 128
    tail_w = c - c_sc
    chunks = []
    c0 = 0
    while c0 < c_sc:
        cc = min(_CC, c_sc - c0)
        chunks.append((c0, cc))
        c0 += cc

    pt, keep = pl.pallas_call(
        functools.partial(_row_stats_kernel, ncls=c),
        grid=(n_tc // br,),
        in_specs=[
            pl.BlockSpec((br, c), lambda i: (i, 0)),
            pl.BlockSpec((br, 1), lambda i: (i, 0)),
        ],
        out_specs=[
            pl.BlockSpec((br, 1), lambda i: (i, 0)),
            pl.BlockSpec((br, 1), lambda i: (i, 0)),
        ],
        out_shape=[
            jax.ShapeDtypeStruct((n_tc, 1), pred.dtype),
            jax.ShapeDtypeStruct((n_tc, 1), pred.dtype),
        ],
    )(pred, tgt2d)

    sc_out = pl.kernel(
        functools.partial(_sc_stats_kernel, n_tc=n_tc, c_sc=c_sc,
                          chunks=tuple(chunks)),
        out_type=jax.ShapeDtypeStruct((_N_SC * 64,), jnp.float32),
        mesh=plsc.VectorSubcoreMesh(core_axis_name="c", subcore_axis_name="s"),
        scratch_types=[
            pltpu.VMEM((_RPW, _CC), jnp.float32),
            pltpu.VMEM((_RPW, _CC), jnp.float32),
            pltpu.VMEM((16,), jnp.int32),
            pltpu.VMEM((_RPW * 64,), jnp.float32),
            pltpu.VMEM((_RPW, 128), jnp.float32),
            pltpu.SemaphoreType.DMA,
            pltpu.SemaphoreType.DMA,
            pltpu.SemaphoreType.DMA,
        ],
    )(pred, tgt)

    tail = pred[n_tc:, c_sc:]
    tm, ts, ttot, txt = pl.pallas_call(
        functools.partial(_tail_stats_kernel, col0=c_sc),
        grid=(1,),
        in_specs=[
            pl.BlockSpec((_N_SC, tail_w), lambda i: (0, 0)),
            pl.BlockSpec((_N_SC, 1), lambda i: (n_tc // _N_SC, 0)),
        ],
        out_specs=[pl.BlockSpec((_N_SC, 1), lambda i: (0, 0))] * 4,
        out_shape=[jax.ShapeDtypeStruct((_N_SC, 1), pred.dtype)] * 4,
    )(tail, tgt2d)

    st = sc_out.reshape(_N_SC, 4, 16)
    m_l, s_l, t_l, xt_l = st[:, 0], st[:, 1], st[:, 2], st[:, 3]
    m_sc = jnp.max(m_l, axis=1)
    s_sc = jnp.sum(s_l * jnp.exp(m_l - m_sc[:, None]), axis=1)
    tot_sc = jnp.sum(t_l, axis=1)
    xt_sc = jnp.sum(xt_l, axis=1)

    tm, ts, ttot, txt = tm[:, 0], ts[:, 0], ttot[:, 0], txt[:, 0]
    m_all = jnp.maximum(m_sc, tm)
    s_all = s_sc * jnp.exp(m_sc - m_all) + ts * jnp.exp(tm - m_all)
    tot_all = tot_sc + ttot
    xt_all = xt_sc + txt

    lse_sc = m_all + jnp.log(s_all)
    fill = _SMOOTHING / (c - 1) if c > 1 else _SMOOTHING
    pt_sc = -(fill * (tot_all - c * lse_sc) + (_CONFIDENCE - fill) * (xt_all - lse_sc))
    keep_sc = (tgt[n_tc:] != _IGNORE_INDEX).astype(pred.dtype)
    pt_sc = pt_sc * keep_sc

    total = jnp.sum(pt) + jnp.sum(pt_sc)
    cnt = jnp.sum(keep) + jnp.sum(keep_sc)
    return total / jnp.maximum(cnt, 1.0)


# speedup vs baseline: 5.6207x; 2.5996x over previous
import functools

import jax
import jax.numpy as jnp
from jax.experimental import pallas as pl

_SMOOTHING = 0.1
_IGNORE_INDEX = 0
_CONFIDENCE = 1.0 - _SMOOTHING

_BC = 2000


def _col_stats_kernel(predt_ref, tgt_ref, m_ref, s_ref, t_ref, xt_ref):
    i = pl.program_id(0)
    x = predt_ref[:, :]
    bc, n = x.shape
    g = 8
    xr = x.reshape(bc // g, g, n)

    @pl.when(i == 0)
    def _init():
        m_ref[:, :] = jnp.full((g, n), -jnp.inf, x.dtype)
        s_ref[:, :] = jnp.zeros((g, n), x.dtype)
        t_ref[:, :] = jnp.zeros((g, n), x.dtype)
        xt_ref[:, :] = jnp.zeros((g, n), x.dtype)

    m_blk = jnp.max(xr, axis=0)
    m_old = m_ref[:, :]
    m_new = jnp.maximum(m_old, m_blk)
    s_new = s_ref[:, :] * jnp.exp(m_old - m_new) + jnp.sum(
        jnp.exp(xr - m_new[None]), axis=0)
    m_ref[:, :] = m_new
    s_ref[:, :] = s_new
    t_ref[:, :] = t_ref[:, :] + jnp.sum(xr, axis=0)

    t = tgt_ref[0, :]
    cls = (i * bc
           + jax.lax.broadcasted_iota(jnp.int32, (bc // g, g, n), 0) * g
           + jax.lax.broadcasted_iota(jnp.int32, (bc // g, g, n), 1))
    xt_ref[:, :] = xt_ref[:, :] + jnp.sum(
        jnp.where(cls == t[None, None, :], xr, 0.0), axis=0)


def kernel(pred, target):
    n, c = pred.shape
    predt = pred.T
    tgt = target.astype(jnp.int32).reshape(1, n)

    g = 8
    m8, s8, t8, xt8 = pl.pallas_call(
        _col_stats_kernel,
        grid=(c // _BC,),
        in_specs=[
            pl.BlockSpec((_BC, n), lambda i: (i, 0)),
            pl.BlockSpec((1, n), lambda i: (0, 0)),
        ],
        out_specs=[pl.BlockSpec((g, n), lambda i: (0, 0))] * 4,
        out_shape=[jax.ShapeDtypeStruct((g, n), pred.dtype)] * 4,
    )(predt, tgt)

    m = jnp.max(m8, axis=0)
    s = jnp.sum(s8 * jnp.exp(m8 - m[None]), axis=0)
    tot = jnp.sum(t8, axis=0)
    xt = jnp.sum(xt8, axis=0)
    lse = m + jnp.log(s)

    fill = _SMOOTHING / (c - 1) if c > 1 else _SMOOTHING
    pt = -(fill * (tot - c * lse) + (_CONFIDENCE - fill) * (xt - lse))
    keep = (tgt[0] != _IGNORE_INDEX).astype(pred.dtype)
    return jnp.sum(pt * keep) / jnp.maximum(jnp.sum(keep), 1.0)


# speedup vs baseline: 5.7652x; 1.0257x over previous
import functools

import jax
import jax.numpy as jnp
from jax import lax
from jax.experimental import pallas as pl
from jax.experimental.pallas import tpu as pltpu
from jax.experimental.pallas import tpu_sc as plsc

_SMOOTHING = 0.1
_IGNORE_INDEX = 0
_CONFIDENCE = 1.0 - _SMOOTHING

_BC = 2400
_SC_CLS = 25600
_CPW = _SC_CLS // 32
_CK = 48


def _col_stats_kernel(predt_ref, m_ref, s_ref, t_ref):
    i = pl.program_id(0)
    x = predt_ref[:, :]
    bc, n = x.shape
    g = 8
    xr = x.reshape(bc // g, g, n)

    @pl.when(i == 0)
    def _init():
        m_ref[:, :] = jnp.full((g, n), -jnp.inf, x.dtype)
        s_ref[:, :] = jnp.zeros((g, n), x.dtype)
        t_ref[:, :] = jnp.zeros((g, n), x.dtype)

    m_blk = jnp.max(xr, axis=0)
    m_old = m_ref[:, :]
    m_new = jnp.maximum(m_old, m_blk)
    s_new = s_ref[:, :] * jnp.exp(m_old - m_new) + jnp.sum(
        jnp.exp(xr - m_new[None]), axis=0)
    m_ref[:, :] = m_new
    s_ref[:, :] = s_new
    t_ref[:, :] = t_ref[:, :] + jnp.sum(xr, axis=0)


def _sc_cls_kernel(predt_hbm, tgt_hbm, out_hbm,
                   buf0, buf1, macc, sacc, tocc, stage, tgt_v, xt_v,
                   sem0, sem1, semg, *, c1, n_tok, chunks):
    wid = lax.axis_index("c") * 16 + lax.axis_index("s")
    cls0 = pl.multiple_of(c1 + wid * _CPW, 8)
    n_chunks = len(chunks)
    n_tg = n_tok // 16
    bufs = (buf0, buf1)
    sems = (sem0, sem1)

    pltpu.sync_copy(tgt_hbm.at[pl.ds(pl.multiple_of(wid * 32, 8), 32)], tgt_v)

    def _start(step):
        o, ck = chunks[step]
        pltpu.make_async_copy(
            predt_hbm.at[pl.ds(cls0 + o, ck), :],
            bufs[step % 2].at[pl.ds(0, ck), :], sems[step % 2]).start()

    iota16 = lax.iota(jnp.int32, 16)
    neg_inf = jnp.full((16,), -jnp.inf, jnp.float32)
    zeros = jnp.zeros((16,), jnp.float32)

    def _init_tg(tg, carry):
        macc[pl.ds(tg * 16, 16)] = neg_inf
        sacc[pl.ds(tg * 16, 16)] = zeros
        tocc[pl.ds(tg * 16, 16)] = zeros
        return carry
    lax.fori_loop(0, n_tg, _init_tg, 0)

    _start(0)
    for step in range(n_chunks):
        o, ck = chunks[step]
        buf = bufs[step % 2]
        pltpu.make_async_copy(
            predt_hbm.at[pl.ds(cls0 + o, ck), :],
            buf.at[pl.ds(0, ck), :], sems[step % 2]).wait()
        if step + 1 < n_chunks:
            _start(step + 1)

        def _do_tg(tg, carry, buf=buf, ck=ck):
            mv = macc[pl.ds(tg * 16, 16)]
            tv = tocc[pl.ds(tg * 16, 16)]

            def _sweep1(i, carry):
                cm, tr = carry
                v = buf[i, pl.ds(tg * 16, 16)]
                return jnp.maximum(cm, v), tr + v
            cm, tv = lax.fori_loop(0, ck, _sweep1, (mv, tv), unroll=4)

            sv = sacc[pl.ds(tg * 16, 16)] * jnp.exp(mv - cm)

            def _sweep2(i, sr):
                v = buf[i, pl.ds(tg * 16, 16)]
                return sr + jnp.exp(v - cm)
            sv = lax.fori_loop(0, ck, _sweep2, sv, unroll=4)

            macc[pl.ds(tg * 16, 16)] = cm
            sacc[pl.ds(tg * 16, 16)] = sv
            tocc[pl.ds(tg * 16, 16)] = tv
            return carry
        lax.fori_loop(0, n_tg, _do_tg, 0)

    tvA = tgt_v[pl.ds(0, 16)]
    tvB = tgt_v[pl.ds(16, 16)]
    xt = [zeros, zeros]
    for j in range(32):
        t_s = (tvA if j < 16 else tvB)[j % 16]
        t8 = pl.multiple_of((t_s // 8) * 8, 8)
        tok = wid * 32 + j
        tile0 = pl.multiple_of((tok // 128) * 128, 128)
        lane = tok - tile0
        l16 = pl.multiple_of((lane // 16) * 16, 16)
        li = lane % 16
        pltpu.make_async_copy(
            predt_hbm.at[pl.ds(t8, 8), pl.ds(tile0, 128)], stage, semg).start()
        pltpu.make_async_copy(
            predt_hbm.at[pl.ds(t8, 8), pl.ds(tile0, 128)], stage, semg).wait()
        rr = t_s - t8
        ev = zeros
        for r in range(8):
            v = stage[r, pl.ds(l16, 16)]
            ev = ev + jnp.where((iota16 == li) & (rr == r), v, 0.0)
        es = jnp.sum(ev)
        xt[j // 16] = jnp.where(iota16 == j % 16, jnp.full((16,), es), xt[j // 16])

    base = wid * (3 * n_tok + 32)
    pltpu.sync_copy(macc, out_hbm.at[pl.ds(base, n_tok)])
    pltpu.sync_copy(sacc, out_hbm.at[pl.ds(base + n_tok, n_tok)])
    pltpu.sync_copy(tocc, out_hbm.at[pl.ds(base + 2 * n_tok, n_tok)])
    xt_v[pl.ds(0, 16)] = xt[0]
    xt_v[pl.ds(16, 16)] = xt[1]
    pltpu.sync_copy(xt_v, out_hbm.at[pl.ds(base + 3 * n_tok, 32)])


def kernel(pred, target):
    n, c = pred.shape
    predt = pred.T
    tgt = target.astype(jnp.int32)

    c1 = c - _SC_CLS
    chunks = []
    o = 0
    while o < _CPW:
        ck = min(_CK, _CPW - o)
        chunks.append((o, ck))
        o += ck

    wlen = 3 * n + 32
    sc_out = pl.kernel(
        functools.partial(_sc_cls_kernel, c1=c1, n_tok=n,
                          chunks=tuple(chunks)),
        out_type=jax.ShapeDtypeStruct((32 * wlen,), jnp.float32),
        mesh=plsc.VectorSubcoreMesh(core_axis_name="c", subcore_axis_name="s"),
        compiler_params=pltpu.CompilerParams(needs_layout_passes=False),
        scratch_types=[
            pltpu.VMEM((_CK, n), jnp.float32),
            pltpu.VMEM((_CK, n), jnp.float32),
            pltpu.VMEM((n,), jnp.float32),
            pltpu.VMEM((n,), jnp.float32),
            pltpu.VMEM((n,), jnp.float32),
            pltpu.VMEM((8, 128), jnp.float32),
            pltpu.VMEM((32,), jnp.int32),
            pltpu.VMEM((32,), jnp.float32),
            pltpu.SemaphoreType.DMA,
            pltpu.SemaphoreType.DMA,
            pltpu.SemaphoreType.DMA,
        ],
    )(predt, tgt)

    g = 8
    m8, s8, t8 = pl.pallas_call(
        _col_stats_kernel,
        grid=(c1 // _BC,),
        in_specs=[pl.BlockSpec((_BC, n), lambda i: (i, 0))],
        out_specs=[pl.BlockSpec((g, n), lambda i: (0, 0))] * 3,
        out_shape=[jax.ShapeDtypeStruct((g, n), pred.dtype)] * 3,
    )(predt)

    sc = sc_out.reshape(32, wlen)
    m_p = sc[:, :n]
    s_p = sc[:, n:2 * n]
    t_p = sc[:, 2 * n:3 * n]
    xt = sc[:, 3 * n:].reshape(-1)

    m_tc = jnp.max(m8, axis=0)
    m_all = jnp.maximum(m_tc, jnp.max(m_p, axis=0))
    s_all = (jnp.sum(s8 * jnp.exp(m8 - m_all[None]), axis=0)
             + jnp.sum(s_p * jnp.exp(m_p - m_all[None]), axis=0))
    tot = jnp.sum(t8, axis=0) + jnp.sum(t_p, axis=0)
    lse = m_all + jnp.log(s_all)

    fill = _SMOOTHING / (c - 1) if c > 1 else _SMOOTHING
    pt = -(fill * (tot - c * lse) + (_CONFIDENCE - fill) * (xt - lse))
    keep = (tgt != _IGNORE_INDEX).astype(pred.dtype)
    return jnp.sum(pt * keep) / jnp.maximum(jnp.sum(keep), 1.0)


# speedup vs baseline: 6.2886x; 1.0908x over previous
import functools

import jax
import jax.numpy as jnp
from jax import lax
from jax.experimental import pallas as pl
from jax.experimental.pallas import tpu as pltpu
from jax.experimental.pallas import tpu_sc as plsc

_SMOOTHING = 0.1
_IGNORE_INDEX = 0
_CONFIDENCE = 1.0 - _SMOOTHING

_BC = 1304
_SC_CLS = 21760
_CPW = _SC_CLS // 32
_CK = 48
_TPS = 3


def _col_stats_kernel(predt_ref, m_ref, s_ref, t_ref):
    i = pl.program_id(0)
    x = predt_ref[:, :]
    bc, n = x.shape
    g = 8
    xr = x.reshape(bc // g, g, n)

    @pl.when(i == 0)
    def _init():
        m_ref[:, :] = jnp.full((g, n), -jnp.inf, x.dtype)
        s_ref[:, :] = jnp.zeros((g, n), x.dtype)
        t_ref[:, :] = jnp.zeros((g, n), x.dtype)

    m_blk = jnp.max(xr, axis=0)
    m_old = m_ref[:, :]
    m_new = jnp.maximum(m_old, m_blk)
    s_new = s_ref[:, :] * jnp.exp(m_old - m_new) + jnp.sum(
        jnp.exp(xr - m_new[None]), axis=0)
    m_ref[:, :] = m_new
    s_ref[:, :] = s_new
    t_ref[:, :] = t_ref[:, :] + jnp.sum(xr, axis=0)


def _sc_cls_kernel(predt_hbm, tgt_hbm, out_hbm,
                   buf0, buf1, macc, sacc, tocc, stage, tgt_v, xt_v,
                   sem0, sem1, semg, *, c1, n_tok, chunks):
    wid = lax.axis_index("c") * 16 + lax.axis_index("s")
    cls0 = pl.multiple_of(c1 + wid * _CPW, 8)
    n_chunks = len(chunks)
    n_tg = n_tok // 16
    bufs = (buf0, buf1)
    sems = (sem0, sem1)

    pltpu.sync_copy(tgt_hbm.at[pl.ds(pl.multiple_of(wid * 32, 8), 32)], tgt_v)
    tvA = tgt_v[pl.ds(0, 16)]
    tvB = tgt_v[pl.ds(16, 16)]

    def _start(step):
        o, ck = chunks[step]
        pltpu.make_async_copy(
            predt_hbm.at[pl.ds(cls0 + o, ck), :],
            bufs[step % 2].at[pl.ds(0, ck), :], sems[step % 2]).start()

    iota16 = lax.iota(jnp.int32, 16)
    neg_inf = jnp.full((16,), -jnp.inf, jnp.float32)
    zeros = jnp.zeros((16,), jnp.float32)

    xt = [zeros, zeros]

    def _tok_params(j):
        t_s = (tvA if j < 16 else tvB)[j % 16]
        t8 = pl.multiple_of((t_s // 8) * 8, 8)
        tok = wid * 32 + j
        tile0 = pl.multiple_of((tok // 128) * 128, 128)
        return t_s, t8, tile0, tok

    def _stage_copy(j, slot):
        t_s, t8, tile0, _ = _tok_params(j)
        return pltpu.make_async_copy(
            predt_hbm.at[pl.ds(t8, 8), pl.ds(tile0, 128)],
            stage.at[pl.ds(slot * 8, 8), :], semg)

    def _stage_fin(j, slot):
        _stage_copy(j, slot).wait()
        t_s, t8, tile0, tok = _tok_params(j)
        lane = tok - tile0
        l16 = pl.multiple_of((lane // 16) * 16, 16)
        li = lane % 16
        rr = t_s - t8
        ev = zeros
        for r in range(8):
            v = stage[slot * 8 + r, pl.ds(l16, 16)]
            ev = ev + jnp.where((iota16 == li) & (rr == r), v, 0.0)
        es = jnp.sum(ev)
        xt[j // 16] = jnp.where(iota16 == j % 16, jnp.full((16,), es),
                                xt[j // 16])

    def _init_tg(tg, carry):
        macc[pl.ds(tg * 16, 16)] = neg_inf
        sacc[pl.ds(tg * 16, 16)] = zeros
        tocc[pl.ds(tg * 16, 16)] = zeros
        return carry
    lax.fori_loop(0, n_tg, _init_tg, 0)

    _start(0)
    for step in range(n_chunks):
        o, ck = chunks[step]
        buf = bufs[step % 2]
        pltpu.make_async_copy(
            predt_hbm.at[pl.ds(cls0 + o, ck), :],
            buf.at[pl.ds(0, ck), :], sems[step % 2]).wait()
        if step + 1 < n_chunks:
            _start(step + 1)
        for k in range(_TPS):
            if step * _TPS + k < 32:
                _stage_copy(step * _TPS + k, k).start()

        def _do_tg(tg, carry, buf=buf, ck=ck):
            mv = macc[pl.ds(tg * 16, 16)]
            tv = tocc[pl.ds(tg * 16, 16)]

            def _sweep1(i, carry):
                cm, tr = carry
                v = buf[i, pl.ds(tg * 16, 16)]
                return jnp.maximum(cm, v), tr + v
            cm, tv = lax.fori_loop(0, ck, _sweep1, (mv, tv), unroll=4)

            sv = sacc[pl.ds(tg * 16, 16)] * jnp.exp(mv - cm)

            def _sweep2(i, sr):
                v = buf[i, pl.ds(tg * 16, 16)]
                return sr + jnp.exp(v - cm)
            sv = lax.fori_loop(0, ck, _sweep2, sv, unroll=4)

            macc[pl.ds(tg * 16, 16)] = cm
            sacc[pl.ds(tg * 16, 16)] = sv
            tocc[pl.ds(tg * 16, 16)] = tv
            return carry
        lax.fori_loop(0, n_tg, _do_tg, 0)

        for k in range(_TPS):
            if step * _TPS + k < 32:
                _stage_fin(step * _TPS + k, k)

    base = wid * (3 * n_tok + 32)
    pltpu.sync_copy(macc, out_hbm.at[pl.ds(base, n_tok)])
    pltpu.sync_copy(sacc, out_hbm.at[pl.ds(base + n_tok, n_tok)])
    pltpu.sync_copy(tocc, out_hbm.at[pl.ds(base + 2 * n_tok, n_tok)])
    xt_v[pl.ds(0, 16)] = xt[0]
    xt_v[pl.ds(16, 16)] = xt[1]
    pltpu.sync_copy(xt_v, out_hbm.at[pl.ds(base + 3 * n_tok, 32)])


def kernel(pred, target):
    n, c = pred.shape
    predt = pred.T
    tgt = target.astype(jnp.int32)

    c1 = c - _SC_CLS
    chunks = []
    o = 0
    while o < _CPW:
        ck = min(_CK, _CPW - o)
        chunks.append((o, ck))
        o += ck

    wlen = 3 * n + 32
    sc_out = pl.kernel(
        functools.partial(_sc_cls_kernel, c1=c1, n_tok=n,
                          chunks=tuple(chunks)),
        out_type=jax.ShapeDtypeStruct((32 * wlen,), jnp.float32),
        mesh=plsc.VectorSubcoreMesh(core_axis_name="c", subcore_axis_name="s"),
        compiler_params=pltpu.CompilerParams(needs_layout_passes=False),
        scratch_types=[
            pltpu.VMEM((_CK, n), jnp.float32),
            pltpu.VMEM((_CK, n), jnp.float32),
            pltpu.VMEM((n,), jnp.float32),
            pltpu.VMEM((n,), jnp.float32),
            pltpu.VMEM((n,), jnp.float32),
            pltpu.VMEM((_TPS * 8, 128), jnp.float32),
            pltpu.VMEM((32,), jnp.int32),
            pltpu.VMEM((32,), jnp.float32),
            pltpu.SemaphoreType.DMA,
            pltpu.SemaphoreType.DMA,
            pltpu.SemaphoreType.DMA,
        ],
    )(predt, tgt)

    g = 8
    m8, s8, t8 = pl.pallas_call(
        _col_stats_kernel,
        grid=(c1 // _BC,),
        in_specs=[pl.BlockSpec((_BC, n), lambda i: (i, 0))],
        out_specs=[pl.BlockSpec((g, n), lambda i: (0, 0))] * 3,
        out_shape=[jax.ShapeDtypeStruct((g, n), pred.dtype)] * 3,
    )(predt)

    sc = sc_out.reshape(32, wlen)
    m_p = sc[:, :n]
    s_p = sc[:, n:2 * n]
    t_p = sc[:, 2 * n:3 * n]
    xt = sc[:, 3 * n:].reshape(-1)

    m_tc = jnp.max(m8, axis=0)
    m_all = jnp.maximum(m_tc, jnp.max(m_p, axis=0))
    s_all = (jnp.sum(s8 * jnp.exp(m8 - m_all[None]), axis=0)
             + jnp.sum(s_p * jnp.exp(m_p - m_all[None]), axis=0))
    tot = jnp.sum(t8, axis=0) + jnp.sum(t_p, axis=0)
    lse = m_all + jnp.log(s_all)

    fill = _SMOOTHING / (c - 1) if c > 1 else _SMOOTHING
    pt = -(fill * (tot - c * lse) + (_CONFIDENCE - fill) * (xt - lse))
    keep = (tgt != _IGNORE_INDEX).astype(pred.dtype)
    return jnp.sum(pt * keep) / jnp.maximum(jnp.sum(keep), 1.0)


# speedup vs baseline: 6.4709x; 1.0290x over previous
import functools

import jax
import jax.numpy as jnp
from jax import lax
from jax.experimental import pallas as pl
from jax.experimental.pallas import tpu as pltpu
from jax.experimental.pallas import tpu_sc as plsc

_SMOOTHING = 0.1
_IGNORE_INDEX = 0
_CONFIDENCE = 1.0 - _SMOOTHING

_BC = 2256
_SC_CLS = 23296
_CPW = _SC_CLS // 32
_CK = 56
_TPS = 3


def _col_stats_kernel(predt_ref, m_ref, s_ref, t_ref):
    i = pl.program_id(0)
    x = predt_ref[:, :]
    bc, n = x.shape
    g = 8
    xr = x.reshape(bc // g, g, n)

    @pl.when(i == 0)
    def _init():
        m_ref[:, :] = jnp.full((g, n), -jnp.inf, x.dtype)
        s_ref[:, :] = jnp.zeros((g, n), x.dtype)
        t_ref[:, :] = jnp.zeros((g, n), x.dtype)

    m_blk = jnp.max(xr, axis=0)
    m_old = m_ref[:, :]
    m_new = jnp.maximum(m_old, m_blk)
    s_new = s_ref[:, :] * jnp.exp(m_old - m_new) + jnp.sum(
        jnp.exp(xr - m_new[None]), axis=0)
    m_ref[:, :] = m_new
    s_ref[:, :] = s_new
    t_ref[:, :] = t_ref[:, :] + jnp.sum(xr, axis=0)


def _sc_cls_kernel(predt_hbm, tgt_hbm, out_hbm,
                   buf0, buf1, macc, sacc, tocc, stage, tgt_v, xt_v,
                   sem0, sem1, semg, *, c1, n_tok, chunks):
    wid = lax.axis_index("c") * 16 + lax.axis_index("s")
    cls0 = pl.multiple_of(c1 + wid * _CPW, 8)
    n_chunks = len(chunks)
    n_tg = n_tok // 16
    bufs = (buf0, buf1)
    sems = (sem0, sem1)

    pltpu.sync_copy(tgt_hbm.at[pl.ds(pl.multiple_of(wid * 32, 8), 32)], tgt_v)
    tvA = tgt_v[pl.ds(0, 16)]
    tvB = tgt_v[pl.ds(16, 16)]

    def _start(step):
        o, ck = chunks[step]
        pltpu.make_async_copy(
            predt_hbm.at[pl.ds(cls0 + o, ck), :],
            bufs[step % 2].at[pl.ds(0, ck), :], sems[step % 2]).start()

    iota16 = lax.iota(jnp.int32, 16)
    neg_inf = jnp.full((16,), -jnp.inf, jnp.float32)
    zeros = jnp.zeros((16,), jnp.float32)

    xt = [zeros, zeros]

    def _tok_params(j):
        t_s = (tvA if j < 16 else tvB)[j % 16]
        t8 = pl.multiple_of((t_s // 8) * 8, 8)
        tok = wid * 32 + j
        tile0 = pl.multiple_of((tok // 128) * 128, 128)
        return t_s, t8, tile0, tok

    def _stage_copy(j, slot):
        t_s, t8, tile0, _ = _tok_params(j)
        return pltpu.make_async_copy(
            predt_hbm.at[pl.ds(t8, 8), pl.ds(tile0, 128)],
            stage.at[pl.ds(slot * 8, 8), :], semg)

    def _stage_fin(j, slot):
        _stage_copy(j, slot).wait()
        t_s, t8, tile0, tok = _tok_params(j)
        lane = tok - tile0
        l16 = pl.multiple_of((lane // 16) * 16, 16)
        li = lane % 16
        rr = t_s - t8
        ev = zeros
        for r in range(8):
            v = stage[slot * 8 + r, pl.ds(l16, 16)]
            ev = ev + jnp.where((iota16 == li) & (rr == r), v, 0.0)
        es = jnp.sum(ev)
        xt[j // 16] = jnp.where(iota16 == j % 16, jnp.full((16,), es),
                                xt[j // 16])

    def _init_tg(tg, carry):
        macc[pl.ds(tg * 16, 16)] = neg_inf
        sacc[pl.ds(tg * 16, 16)] = zeros
        tocc[pl.ds(tg * 16, 16)] = zeros
        return carry
    lax.fori_loop(0, n_tg, _init_tg, 0)

    _start(0)
    for step in range(n_chunks):
        o, ck = chunks[step]
        buf = bufs[step % 2]
        pltpu.make_async_copy(
            predt_hbm.at[pl.ds(cls0 + o, ck), :],
            buf.at[pl.ds(0, ck), :], sems[step % 2]).wait()
        if step + 1 < n_chunks:
            _start(step + 1)
        for k in range(_TPS):
            if step * _TPS + k < 32:
                _stage_copy(step * _TPS + k, k).start()

        def _do_tg(tg, carry, buf=buf, ck=ck):
            mv = macc[pl.ds(tg * 16, 16)]
            tv = tocc[pl.ds(tg * 16, 16)]

            def _sweep1(i, carry):
                cm, tr = carry
                v = buf[i, pl.ds(tg * 16, 16)]
                return jnp.maximum(cm, v), tr + v
            cm, tv = lax.fori_loop(0, ck, _sweep1, (mv, tv), unroll=8)

            sv = sacc[pl.ds(tg * 16, 16)] * jnp.exp(mv - cm)

            def _sweep2(i, sr):
                v = buf[i, pl.ds(tg * 16, 16)]
                return sr + jnp.exp(v - cm)
            sv = lax.fori_loop(0, ck, _sweep2, sv, unroll=8)

            macc[pl.ds(tg * 16, 16)] = cm
            sacc[pl.ds(tg * 16, 16)] = sv
            tocc[pl.ds(tg * 16, 16)] = tv
            return carry
        lax.fori_loop(0, n_tg, _do_tg, 0)

        for k in range(_TPS):
            if step * _TPS + k < 32:
                _stage_fin(step * _TPS + k, k)

    base = wid * (3 * n_tok + 32)
    pltpu.sync_copy(macc, out_hbm.at[pl.ds(base, n_tok)])
    pltpu.sync_copy(sacc, out_hbm.at[pl.ds(base + n_tok, n_tok)])
    pltpu.sync_copy(tocc, out_hbm.at[pl.ds(base + 2 * n_tok, n_tok)])
    xt_v[pl.ds(0, 16)] = xt[0]
    xt_v[pl.ds(16, 16)] = xt[1]
    pltpu.sync_copy(xt_v, out_hbm.at[pl.ds(base + 3 * n_tok, 32)])


def kernel(pred, target):
    n, c = pred.shape
    predt = pred.T
    tgt = target.astype(jnp.int32)

    c1 = c - _SC_CLS
    chunks = []
    o = 0
    while o < _CPW:
        ck = min(_CK, _CPW - o)
        chunks.append((o, ck))
        o += ck

    wlen = 3 * n + 32
    sc_out = pl.kernel(
        functools.partial(_sc_cls_kernel, c1=c1, n_tok=n,
                          chunks=tuple(chunks)),
        out_type=jax.ShapeDtypeStruct((32 * wlen,), jnp.float32),
        mesh=plsc.VectorSubcoreMesh(core_axis_name="c", subcore_axis_name="s"),
        compiler_params=pltpu.CompilerParams(needs_layout_passes=False),
        scratch_types=[
            pltpu.VMEM((_CK, n), jnp.float32),
            pltpu.VMEM((_CK, n), jnp.float32),
            pltpu.VMEM((n,), jnp.float32),
            pltpu.VMEM((n,), jnp.float32),
            pltpu.VMEM((n,), jnp.float32),
            pltpu.VMEM((_TPS * 8, 128), jnp.float32),
            pltpu.VMEM((32,), jnp.int32),
            pltpu.VMEM((32,), jnp.float32),
            pltpu.SemaphoreType.DMA,
            pltpu.SemaphoreType.DMA,
            pltpu.SemaphoreType.DMA,
        ],
    )(predt, tgt)

    g = 8
    m8, s8, t8 = pl.pallas_call(
        _col_stats_kernel,
        grid=(c1 // _BC,),
        in_specs=[pl.BlockSpec((_BC, n), lambda i: (i, 0))],
        out_specs=[pl.BlockSpec((g, n), lambda i: (0, 0))] * 3,
        out_shape=[jax.ShapeDtypeStruct((g, n), pred.dtype)] * 3,
    )(predt)

    sc = sc_out.reshape(32, wlen)
    m_p = sc[:, :n]
    s_p = sc[:, n:2 * n]
    t_p = sc[:, 2 * n:3 * n]
    xt = sc[:, 3 * n:].reshape(-1)

    m_tc = jnp.max(m8, axis=0)
    m_all = jnp.maximum(m_tc, jnp.max(m_p, axis=0))
    s_all = (jnp.sum(s8 * jnp.exp(m8 - m_all[None]), axis=0)
             + jnp.sum(s_p * jnp.exp(m_p - m_all[None]), axis=0))
    tot = jnp.sum(t8, axis=0) + jnp.sum(t_p, axis=0)
    lse = m_all + jnp.log(s_all)

    fill = _SMOOTHING / (c - 1) if c > 1 else _SMOOTHING
    pt = -(fill * (tot - c * lse) + (_CONFIDENCE - fill) * (xt - lse))
    keep = (tgt != _IGNORE_INDEX).astype(pred.dtype)
    return jnp.sum(pt * keep) / jnp.maximum(jnp.sum(keep), 1.0)
